# baseline (speedup 1.0000x reference)
import functools

import jax
import jax.numpy as jnp
from jax import lax
from jax.experimental import pallas as pl
from jax.experimental.pallas import tpu as pltpu
from jax.experimental.pallas import tpu_sc as plsc

D_MODEL = 256
N_HEADS = 8
HEAD_DIM = D_MODEL // N_HEADS
D_FF = 1024
ROW_BLK = 400

NC = 2
NS = 16
L = 16
NW = NC * NS

N_NODES = 10000
N_EDGES = 160000
CH_A = 128
CHUNKS_A = 40
E_PAD = NW * CHUNKS_A * CH_A
NODE_BLK = 313
SCH = 640
NCHUNK_B = N_EDGES // SCH
PENDCAP = 128
FLUSH_AT = 112
INV_SQRT_C = 0.17677669529663687


def _qkvs_body(x_ref, w_ref, b_ref, q_ref, k_ref, v_ref, s_ref):
    y = (
        jnp.dot(x_ref[...], w_ref[...], preferred_element_type=jnp.float32)
        + b_ref[...]
    )
    q_ref[...] = y[:, 0:D_MODEL]
    k_ref[...] = y[:, D_MODEL:2 * D_MODEL]
    v_ref[...] = y[:, 2 * D_MODEL:3 * D_MODEL]
    s_ref[...] = y[:, 3 * D_MODEL:4 * D_MODEL]


def _qkvs(node, Wcat, bcat):
    n = node.shape[0]
    grid = n // ROW_BLK
    row = lambda i: (i, 0)
    out = jax.ShapeDtypeStruct((n, D_MODEL), jnp.float32)
    return pl.pallas_call(
        _qkvs_body,
        grid=(grid,),
        in_specs=[
            pl.BlockSpec((ROW_BLK, D_MODEL), row),
            pl.BlockSpec((D_MODEL, 4 * D_MODEL), lambda i: (0, 0)),
            pl.BlockSpec((1, 4 * D_MODEL), lambda i: (0, 0)),
        ],
        out_specs=[pl.BlockSpec((ROW_BLK, D_MODEL), row)] * 4,
        out_shape=[out, out, out, out],
    )(node, Wcat, bcat)


def _ln(x, g, b, eps=1e-5):
    mu = jnp.mean(x, axis=-1, keepdims=True)
    var = jnp.mean((x - mu) ** 2, axis=-1, keepdims=True)
    return (x - mu) * lax.rsqrt(var + eps) * g + b


def _post_body(node_ref, agg_ref, skip_ref, wout_ref, bout_ref, g1_ref, be1_ref,
               w1_ref, b1_ref, w2_ref, b2_ref, g2_ref, be2_ref, o_ref):
    attn = agg_ref[...] + skip_ref[...]
    y = jnp.dot(attn, wout_ref[...], preferred_element_type=jnp.float32) + bout_ref[...]
    x1 = _ln(node_ref[...] + y, g1_ref[...], be1_ref[...])
    h = jnp.maximum(
        jnp.dot(x1, w1_ref[...], preferred_element_type=jnp.float32) + b1_ref[...],
        0.0,
    )
    ff = jnp.dot(h, w2_ref[...], preferred_element_type=jnp.float32) + b2_ref[...]
    o_ref[...] = _ln(x1 + ff, g2_ref[...], be2_ref[...])


def _post(node, agg, skip, Wout, bout, g1, be1, W1, b1, W2, b2, g2, be2):
    n = node.shape[0]
    grid = n // ROW_BLK
    row = lambda i: (i, 0)
    full = lambda i: (0, 0)
    return pl.pallas_call(
        _post_body,
        grid=(grid,),
        in_specs=[
            pl.BlockSpec((ROW_BLK, D_MODEL), row),
            pl.BlockSpec((ROW_BLK, D_MODEL), row),
            pl.BlockSpec((ROW_BLK, D_MODEL), row),
            pl.BlockSpec((D_MODEL, D_MODEL), full),
            pl.BlockSpec((1, D_MODEL), full),
            pl.BlockSpec((1, D_MODEL), full),
            pl.BlockSpec((1, D_MODEL), full),
            pl.BlockSpec((D_MODEL, D_FF), full),
            pl.BlockSpec((1, D_FF), full),
            pl.BlockSpec((D_FF, D_MODEL), full),
            pl.BlockSpec((1, D_MODEL), full),
            pl.BlockSpec((1, D_MODEL), full),
            pl.BlockSpec((1, D_MODEL), full),
        ],
        out_specs=pl.BlockSpec((ROW_BLK, D_MODEL), row),
        out_shape=jax.ShapeDtypeStruct((n, D_MODEL), jnp.float32),
    )(node, agg, skip, Wout, bout, g1, be1, W1, b1, W2, b2, g2, be2)


def _edge_logits(q, k, dst_a, src_a):
    mesh = plsc.VectorSubcoreMesh(core_axis_name="c", subcore_axis_name="s")

    @functools.partial(
        pl.kernel,
        mesh=mesh,
        compiler_params=pltpu.CompilerParams(use_tc_tiling_on_sc=False,
                                             needs_layout_passes=False),
        out_type=jax.ShapeDtypeStruct((E_PAD, 16), jnp.float32),
        scratch_types=[
            pltpu.VMEM((CH_A,), jnp.int32),
            pltpu.VMEM((CH_A,), jnp.int32),
            pltpu.VMEM((CH_A, D_MODEL), jnp.float32),
            pltpu.VMEM((CH_A, D_MODEL), jnp.float32),
            pltpu.VMEM((CH_A, 16), jnp.float32),
            pltpu.SemaphoreType.DMA,
            pltpu.SemaphoreType.DMA,
        ],
    )
    def kern(q_hbm, k_hbm, dst_hbm, src_hbm, p_hbm,
             dstb, srcb, qbuf, kbuf, pbuf, sem1, sem2):
        wid = lax.axis_index("s") * NC + lax.axis_index("c")
        lanes = lax.iota(jnp.int32, L)
        zero16 = jnp.zeros((L,), jnp.float32)

        def chunk_body(ci, carry):
            base = (wid * CHUNKS_A + ci) * CH_A
            pltpu.sync_copy(dst_hbm.at[pl.ds(base, CH_A)], dstb)
            pltpu.sync_copy(src_hbm.at[pl.ds(base, CH_A)], srcb)
            c1 = pltpu.async_copy(q_hbm.at[dstb], qbuf, sem1)
            c2 = pltpu.async_copy(k_hbm.at[srcb], kbuf, sem2)
            c1.wait()
            c2.wait()
            for g in range(CH_A // L):
                rows = g * L + lanes
                for h in range(N_HEADS):
                    def col_body(cc, acc):
                        for u in range(8):
                            colv = jnp.full((L,), h * HEAD_DIM, jnp.int32) + (cc * 8 + u)
                            qc = plsc.load_gather(qbuf, [rows, colv])
                            kc = plsc.load_gather(kbuf, [rows, colv])
                            acc = acc + qc * kc
                        return acc
                    acc = lax.fori_loop(0, HEAD_DIM // 8, col_body,
                                        jnp.zeros((L,), jnp.float32))
                    pv = jnp.exp(acc * INV_SQRT_C)
                    plsc.store_scatter(
                        pbuf, [rows, jnp.full((L,), h, jnp.int32)], pv)
                for h in range(N_HEADS, 16):
                    plsc.store_scatter(
                        pbuf, [rows, jnp.full((L,), h, jnp.int32)], zero16)
            pltpu.sync_copy(pbuf, p_hbm.at[pl.ds(base, CH_A)])
            return carry

        lax.fori_loop(0, CHUNKS_A, chunk_body, 0)

    return kern(q, k, dst_a, src_a)


def _edge_aggregate(v, p, dst, src):
    mesh = plsc.VectorSubcoreMesh(core_axis_name="c", subcore_axis_name="s")

    @functools.partial(
        pl.kernel,
        mesh=mesh,
        compiler_params=pltpu.CompilerParams(use_tc_tiling_on_sc=False,
                                             needs_layout_passes=False),
        out_type=jax.ShapeDtypeStruct((N_NODES, D_MODEL), jnp.float32),
        scratch_types=[
            pltpu.VMEM((NODE_BLK, D_MODEL), jnp.float32),
            pltpu.VMEM((NODE_BLK, 16), jnp.float32),
            pltpu.VMEM((PENDCAP, D_MODEL), jnp.float32),
            pltpu.VMEM((PENDCAP, 16), jnp.float32),
            pltpu.VMEM((PENDCAP,), jnp.int32),
            pltpu.VMEM((PENDCAP,), jnp.int32),
            pltpu.VMEM((PENDCAP + L,), jnp.int32),
            pltpu.VMEM((SCH,), jnp.int32),
            pltpu.VMEM((SCH,), jnp.int32),
            pltpu.SemaphoreType.DMA,
            pltpu.SemaphoreType.DMA,
        ],
    )
    def kern(v_hbm, p_hbm, dst_hbm, src_hbm, agg_hbm,
             aggv, sv, vbuf, pbv, pend_src, pend_e, pend_dl,
             dstc, srcc, sem1, sem2):
        wid = lax.axis_index("s") * NC + lax.axis_index("c")
        lanes = lax.iota(jnp.int32, L)
        lo = wid * NODE_BLK
        hi = lo + NODE_BLK
        zero16f = jnp.zeros((L,), jnp.float32)
        zero16i = jnp.zeros((L,), jnp.int32)

        def z_body(r, carry):
            for half in range(D_MODEL // L):
                aggv[r, pl.ds(half * L, L)] = zero16f
            sv[r, pl.ds(0, L)] = zero16f
            return carry
        lax.fori_loop(0, NODE_BLK, z_body, 0)
        for i in range(PENDCAP // L):
            pend_src[pl.ds(i * L, L)] = zero16i
            pend_e[pl.ds(i * L, L)] = zero16i

        def flush(cnt):
            cv = pltpu.async_copy(v_hbm.at[pend_src], vbuf, sem1)
            cp = pltpu.async_copy(p_hbm.at[pend_e], pbv, sem2)
            cv.wait()
            cp.wait()

            def acc_body(i, carry):
                dl = pend_dl[pl.ds(i, L)][0]
                prow = pbv[i]
                plsc.addupdate(sv.at[dl], prow)
                for h in range(N_HEADS):
                    ps = prow[h]
                    for half in range(2):
                        c0 = h * HEAD_DIM + half * L
                        vv = vbuf[i, pl.ds(c0, L)]
                        plsc.addupdate(aggv.at[dl, pl.ds(c0, L)], vv * ps)
                return carry

            lax.fori_loop(0, cnt, acc_body, 0)

        def chunk_body(ci, off):
            cb = ci * SCH
            pltpu.sync_copy(dst_hbm.at[pl.ds(cb, SCH)], dstc)
            pltpu.sync_copy(src_hbm.at[pl.ds(cb, SCH)], srcc)

            def group_body(gi, off):
                gb = gi * L
                dstv = dstc[pl.ds(gb, L)]
                srcv = srcc[pl.ds(gb, L)]
                m = (dstv >= lo) & (dstv < hi)
                cnt = plsc.all_reduce_population_count(m)[0]

                def do_flush(o):
                    flush(o)
                    return 0
                off = lax.cond(off + L > PENDCAP, do_flush, lambda o: o, off)
                plsc.store_compressed(pend_src.at[pl.ds(off, L)], srcv, mask=m)
                plsc.store_compressed(pend_e.at[pl.ds(off, L)],
                                      cb + gb + lanes, mask=m)
                plsc.store_compressed(pend_dl.at[pl.ds(off, L)],
                                      dstv - lo, mask=m)
                return off + cnt

            return lax.fori_loop(0, SCH // L, group_body, off)

        off = lax.fori_loop(0, NCHUNK_B, chunk_body, 0)
        lax.cond(off > 0, lambda o: (flush(o), 0)[1], lambda o: 0, off)

        def norm_body(r, carry):
            invrow = 1.0 / (sv[r] + 1e-16)
            for h in range(N_HEADS):
                inv = invrow[h]
                for half in range(2):
                    c0 = h * HEAD_DIM + half * L
                    aggv[r, pl.ds(c0, L)] = aggv[r, pl.ds(c0, L)] * inv
            return carry
        lax.fori_loop(0, NODE_BLK, norm_body, 0)

        @pl.when(wid < NW - 1)
        def _():
            pltpu.sync_copy(aggv, agg_hbm.at[pl.ds(lo, NODE_BLK)])

        @pl.when(wid == NW - 1)
        def _():
            tail = N_NODES - (NW - 1) * NODE_BLK
            pltpu.sync_copy(aggv.at[pl.ds(0, tail)],
                            agg_hbm.at[pl.ds(lo, tail)])

    return kern(v, p, dst, src)


def kernel(node, edge_index, Wq, bq, Wk, bk, Wv, bv, Wskip, bskip, Wout, bout,
           g1, be1, W1, b1, W2, b2, g2, be2):
    n = node.shape[0]
    src = edge_index[0].astype(jnp.int32)
    dst = edge_index[1].astype(jnp.int32)
    Wcat = jnp.concatenate([Wq, Wk, Wv, Wskip], axis=1)
    bcat = jnp.concatenate([bq, bk, bv, bskip]).reshape(1, 4 * D_MODEL)
    q, k, v, skip = _qkvs(node, Wcat, bcat)

    pad_e = E_PAD - N_EDGES
    src_a = jnp.pad(src, (0, pad_e))
    dst_a = jnp.pad(dst, (0, pad_e))
    p = _edge_logits(q, k, dst_a, src_a)
    agg = _edge_aggregate(v, p, dst, src)

    return _post(node, agg, skip, Wout, bout.reshape(1, -1), g1.reshape(1, -1),
                 be1.reshape(1, -1), W1, b1.reshape(1, -1), W2, b2.reshape(1, -1),
                 g2.reshape(1, -1), be2.reshape(1, -1))


# speedup vs baseline: 1.1558x; 1.1558x over previous
import functools

import jax
import jax.numpy as jnp
from jax import lax
from jax.experimental import pallas as pl
from jax.experimental.pallas import tpu as pltpu
from jax.experimental.pallas import tpu_sc as plsc

D_MODEL = 256
N_HEADS = 8
HEAD_DIM = D_MODEL // N_HEADS
D_FF = 1024
ROW_BLK = 400

NC = 2
NS = 16
L = 16
NW = NC * NS

N_NODES = 10000
N_EDGES = 160000
CH_A = 128
CHUNKS_A = 40
E_PAD = NW * CHUNKS_A * CH_A
NODE_BLK = 313
SCH = 640
NCHUNK_B = N_EDGES // SCH
PENDCAP = 128
FLUSH_AT = 112
INV_SQRT_C = 0.17677669529663687


def _qkvs_body(x_ref, w_ref, b_ref, q_ref, k_ref, v_ref, s_ref):
    y = (
        jnp.dot(x_ref[...], w_ref[...], preferred_element_type=jnp.float32)
        + b_ref[...]
    )
    q_ref[...] = y[:, 0:D_MODEL]
    k_ref[...] = y[:, D_MODEL:2 * D_MODEL]
    v_ref[...] = y[:, 2 * D_MODEL:3 * D_MODEL]
    s_ref[...] = y[:, 3 * D_MODEL:4 * D_MODEL]


def _qkvs(node, Wcat, bcat):
    n = node.shape[0]
    grid = n // ROW_BLK
    row = lambda i: (i, 0)
    out = jax.ShapeDtypeStruct((n, D_MODEL), jnp.float32)
    return pl.pallas_call(
        _qkvs_body,
        grid=(grid,),
        in_specs=[
            pl.BlockSpec((ROW_BLK, D_MODEL), row),
            pl.BlockSpec((D_MODEL, 4 * D_MODEL), lambda i: (0, 0)),
            pl.BlockSpec((1, 4 * D_MODEL), lambda i: (0, 0)),
        ],
        out_specs=[pl.BlockSpec((ROW_BLK, D_MODEL), row)] * 4,
        out_shape=[out, out, out, out],
    )(node, Wcat, bcat)


def _ln(x, g, b, eps=1e-5):
    mu = jnp.mean(x, axis=-1, keepdims=True)
    var = jnp.mean((x - mu) ** 2, axis=-1, keepdims=True)
    return (x - mu) * lax.rsqrt(var + eps) * g + b


def _post_body(node_ref, agg_ref, skip_ref, wout_ref, bout_ref, g1_ref, be1_ref,
               w1_ref, b1_ref, w2_ref, b2_ref, g2_ref, be2_ref, o_ref):
    attn = agg_ref[...] + skip_ref[...]
    y = jnp.dot(attn, wout_ref[...], preferred_element_type=jnp.float32) + bout_ref[...]
    x1 = _ln(node_ref[...] + y, g1_ref[...], be1_ref[...])
    h = jnp.maximum(
        jnp.dot(x1, w1_ref[...], preferred_element_type=jnp.float32) + b1_ref[...],
        0.0,
    )
    ff = jnp.dot(h, w2_ref[...], preferred_element_type=jnp.float32) + b2_ref[...]
    o_ref[...] = _ln(x1 + ff, g2_ref[...], be2_ref[...])


def _post(node, agg, skip, Wout, bout, g1, be1, W1, b1, W2, b2, g2, be2):
    n = node.shape[0]
    grid = n // ROW_BLK
    row = lambda i: (i, 0)
    full = lambda i: (0, 0)
    return pl.pallas_call(
        _post_body,
        grid=(grid,),
        in_specs=[
            pl.BlockSpec((ROW_BLK, D_MODEL), row),
            pl.BlockSpec((ROW_BLK, D_MODEL), row),
            pl.BlockSpec((ROW_BLK, D_MODEL), row),
            pl.BlockSpec((D_MODEL, D_MODEL), full),
            pl.BlockSpec((1, D_MODEL), full),
            pl.BlockSpec((1, D_MODEL), full),
            pl.BlockSpec((1, D_MODEL), full),
            pl.BlockSpec((D_MODEL, D_FF), full),
            pl.BlockSpec((1, D_FF), full),
            pl.BlockSpec((D_FF, D_MODEL), full),
            pl.BlockSpec((1, D_MODEL), full),
            pl.BlockSpec((1, D_MODEL), full),
            pl.BlockSpec((1, D_MODEL), full),
        ],
        out_specs=pl.BlockSpec((ROW_BLK, D_MODEL), row),
        out_shape=jax.ShapeDtypeStruct((n, D_MODEL), jnp.float32),
    )(node, agg, skip, Wout, bout, g1, be1, W1, b1, W2, b2, g2, be2)


def _edge_logits(q, k, dst_a, src_a):
    mesh = plsc.VectorSubcoreMesh(core_axis_name="c", subcore_axis_name="s")

    HW = D_MODEL // 2
    HCOL = HW // N_HEADS

    @functools.partial(
        pl.kernel,
        mesh=mesh,
        compiler_params=pltpu.CompilerParams(use_tc_tiling_on_sc=False,
                                             needs_layout_passes=False),
        out_type=jax.ShapeDtypeStruct((E_PAD, 16), jnp.float32),
        scratch_types=[
            pltpu.VMEM((CH_A,), jnp.int32),
            pltpu.VMEM((CH_A,), jnp.int32),
            pltpu.VMEM((CH_A, HW), jnp.int32),
            pltpu.VMEM((CH_A, HW), jnp.int32),
            pltpu.VMEM((CH_A, 16), jnp.float32),
            pltpu.SemaphoreType.DMA,
            pltpu.SemaphoreType.DMA,
        ],
    )
    def kern(q_hbm, k_hbm, dst_hbm, src_hbm, p_hbm,
             dstb, srcb, qbuf, kbuf, pbuf, sem1, sem2):
        wid = lax.axis_index("s") * NC + lax.axis_index("c")
        lanes = lax.iota(jnp.int32, L)
        zero16 = jnp.zeros((L,), jnp.float32)
        zero_bf = jnp.zeros((2 * L,), jnp.bfloat16)
        himask = jnp.int32(-65536)

        def chunk_body(ci, carry):
            base = (wid * CHUNKS_A + ci) * CH_A
            pltpu.sync_copy(dst_hbm.at[pl.ds(base, CH_A)], dstb)
            pltpu.sync_copy(src_hbm.at[pl.ds(base, CH_A)], srcb)
            c1 = pltpu.async_copy(q_hbm.at[dstb], qbuf, sem1)
            c2 = pltpu.async_copy(k_hbm.at[srcb], kbuf, sem2)
            c1.wait()
            c2.wait()
            for g in range(CH_A // L):
                rows = g * L + lanes
                for h in range(N_HEADS):
                    hbase = jnp.full((L,), h * HCOL, jnp.int32)

                    def col_body(cc, acc):
                        acc_bf = zero_bf
                        for u in range(8):
                            colv = hbase + (cc * 8 + u)
                            qw = plsc.load_gather(qbuf, [rows, colv])
                            kw = plsc.load_gather(kbuf, [rows, colv])
                            acc_bf = acc_bf + (plsc.bitcast(qw, jnp.bfloat16)
                                               * plsc.bitcast(kw, jnp.bfloat16))
                        w = plsc.bitcast(acc_bf, jnp.int32)
                        hi = plsc.bitcast(w & himask, jnp.float32)
                        lo = plsc.bitcast(lax.shift_left(w, 16), jnp.float32)
                        return acc + hi + lo

                    acc = lax.fori_loop(0, HCOL // 8, col_body,
                                        jnp.zeros((L,), jnp.float32))
                    pv = jnp.exp(acc * INV_SQRT_C)
                    plsc.store_scatter(
                        pbuf, [rows, jnp.full((L,), h, jnp.int32)], pv)
                for h in range(N_HEADS, 16):
                    plsc.store_scatter(
                        pbuf, [rows, jnp.full((L,), h, jnp.int32)], zero16)
            pltpu.sync_copy(pbuf, p_hbm.at[pl.ds(base, CH_A)])
            return carry

        lax.fori_loop(0, CHUNKS_A, chunk_body, 0)

    return kern(q, k, dst_a, src_a)


def _edge_aggregate(v, p, dst, src):
    mesh = plsc.VectorSubcoreMesh(core_axis_name="c", subcore_axis_name="s")

    @functools.partial(
        pl.kernel,
        mesh=mesh,
        compiler_params=pltpu.CompilerParams(use_tc_tiling_on_sc=False,
                                             needs_layout_passes=False),
        out_type=jax.ShapeDtypeStruct((N_NODES, D_MODEL), jnp.float32),
        scratch_types=[
            pltpu.VMEM((NODE_BLK, D_MODEL), jnp.float32),
            pltpu.VMEM((NODE_BLK, 16), jnp.float32),
            pltpu.VMEM((PENDCAP, D_MODEL), jnp.float32),
            pltpu.VMEM((PENDCAP, 16), jnp.float32),
            pltpu.VMEM((PENDCAP,), jnp.int32),
            pltpu.VMEM((PENDCAP,), jnp.int32),
            pltpu.VMEM((PENDCAP + L,), jnp.int32),
            pltpu.VMEM((SCH,), jnp.int32),
            pltpu.VMEM((SCH,), jnp.int32),
            pltpu.SemaphoreType.DMA,
            pltpu.SemaphoreType.DMA,
        ],
    )
    def kern(v_hbm, p_hbm, dst_hbm, src_hbm, agg_hbm,
             aggv, sv, vbuf, pbv, pend_src, pend_e, pend_dl,
             dstc, srcc, sem1, sem2):
        wid = lax.axis_index("s") * NC + lax.axis_index("c")
        lanes = lax.iota(jnp.int32, L)
        lo = wid * NODE_BLK
        hi = lo + NODE_BLK
        zero16f = jnp.zeros((L,), jnp.float32)
        zero16i = jnp.zeros((L,), jnp.int32)

        def z_body(r, carry):
            for half in range(D_MODEL // L):
                aggv[r, pl.ds(half * L, L)] = zero16f
            sv[r, pl.ds(0, L)] = zero16f
            return carry
        lax.fori_loop(0, NODE_BLK, z_body, 0)
        for i in range(PENDCAP // L):
            pend_src[pl.ds(i * L, L)] = zero16i
            pend_e[pl.ds(i * L, L)] = zero16i

        def flush(cnt):
            cv = pltpu.async_copy(v_hbm.at[pend_src], vbuf, sem1)
            cp = pltpu.async_copy(p_hbm.at[pend_e], pbv, sem2)
            cv.wait()
            cp.wait()

            def acc_body(i, carry):
                dl = pend_dl[pl.ds(i, L)][0]
                prow = pbv[i]
                plsc.addupdate(sv.at[dl], prow)
                for h in range(N_HEADS):
                    ps = prow[h]
                    for half in range(2):
                        c0 = h * HEAD_DIM + half * L
                        vv = vbuf[i, pl.ds(c0, L)]
                        plsc.addupdate(aggv.at[dl, pl.ds(c0, L)], vv * ps)
                return carry

            lax.fori_loop(0, cnt, acc_body, 0)

        def chunk_body(ci, off):
            cb = ci * SCH
            pltpu.sync_copy(dst_hbm.at[pl.ds(cb, SCH)], dstc)
            pltpu.sync_copy(src_hbm.at[pl.ds(cb, SCH)], srcc)

            def group_body(gi, off):
                gb = gi * L
                dstv = dstc[pl.ds(gb, L)]
                srcv = srcc[pl.ds(gb, L)]
                m = (dstv >= lo) & (dstv < hi)
                cnt = plsc.all_reduce_population_count(m)[0]

                def do_flush(o):
                    flush(o)
                    return 0
                off = lax.cond(off + L > PENDCAP, do_flush, lambda o: o, off)
                plsc.store_compressed(pend_src.at[pl.ds(off, L)], srcv, mask=m)
                plsc.store_compressed(pend_e.at[pl.ds(off, L)],
                                      cb + gb + lanes, mask=m)
                plsc.store_compressed(pend_dl.at[pl.ds(off, L)],
                                      dstv - lo, mask=m)
                return off + cnt

            return lax.fori_loop(0, SCH // L, group_body, off)

        off = lax.fori_loop(0, NCHUNK_B, chunk_body, 0)
        lax.cond(off > 0, lambda o: (flush(o), 0)[1], lambda o: 0, off)

        def norm_body(r, carry):
            invrow = 1.0 / (sv[r] + 1e-16)
            for h in range(N_HEADS):
                inv = invrow[h]
                for half in range(2):
                    c0 = h * HEAD_DIM + half * L
                    aggv[r, pl.ds(c0, L)] = aggv[r, pl.ds(c0, L)] * inv
            return carry
        lax.fori_loop(0, NODE_BLK, norm_body, 0)

        @pl.when(wid < NW - 1)
        def _():
            pltpu.sync_copy(aggv, agg_hbm.at[pl.ds(lo, NODE_BLK)])

        @pl.when(wid == NW - 1)
        def _():
            tail = N_NODES - (NW - 1) * NODE_BLK
            pltpu.sync_copy(aggv.at[pl.ds(0, tail)],
                            agg_hbm.at[pl.ds(lo, tail)])

    return kern(v, p, dst, src)


def kernel(node, edge_index, Wq, bq, Wk, bk, Wv, bv, Wskip, bskip, Wout, bout,
           g1, be1, W1, b1, W2, b2, g2, be2):
    n = node.shape[0]
    src = edge_index[0].astype(jnp.int32)
    dst = edge_index[1].astype(jnp.int32)
    Wcat = jnp.concatenate([Wq, Wk, Wv, Wskip], axis=1)
    bcat = jnp.concatenate([bq, bk, bv, bskip]).reshape(1, 4 * D_MODEL)
    q, k, v, skip = _qkvs(node, Wcat, bcat)

    pad_e = E_PAD - N_EDGES
    src_a = jnp.pad(src, (0, pad_e))
    dst_a = jnp.pad(dst, (0, pad_e))
    q_i32 = lax.bitcast_convert_type(
        q.astype(jnp.bfloat16).reshape(n, D_MODEL // 2, 2), jnp.int32)
    k_i32 = lax.bitcast_convert_type(
        k.astype(jnp.bfloat16).reshape(n, D_MODEL // 2, 2), jnp.int32)
    p = _edge_logits(q_i32, k_i32, dst_a, src_a)
    agg = _edge_aggregate(v, p, dst, src)

    return _post(node, agg, skip, Wout, bout.reshape(1, -1), g1.reshape(1, -1),
                 be1.reshape(1, -1), W1, b1.reshape(1, -1), W2, b2.reshape(1, -1),
                 g2.reshape(1, -1), be2.reshape(1, -1))


# speedup vs baseline: 1.2969x; 1.1221x over previous
import functools

import jax
import jax.numpy as jnp
from jax import lax
from jax.experimental import pallas as pl
from jax.experimental.pallas import tpu as pltpu
from jax.experimental.pallas import tpu_sc as plsc

D_MODEL = 256
N_HEADS = 8
HEAD_DIM = D_MODEL // N_HEADS
D_FF = 1024
ROW_BLK = 400

NC = 2
NS = 16
L = 16
NW = NC * NS

N_NODES = 10000
N_EDGES = 160000
CH_A = 128
CHUNKS_A = 40
E_PAD = NW * CHUNKS_A * CH_A
NODE_BLK = 313
SCH = 640
NCHUNK_B = N_EDGES // SCH
PENDCAP = 128
FLUSH_AT = 112
INV_SQRT_C = 0.17677669529663687


def _qkvs_body(x_ref, w_ref, b_ref, q_ref, k_ref, v_ref, s_ref):
    y = (
        jnp.dot(x_ref[...], w_ref[...], preferred_element_type=jnp.float32)
        + b_ref[...]
    )
    q_ref[...] = y[:, 0:D_MODEL]
    k_ref[...] = y[:, D_MODEL:2 * D_MODEL]
    v_ref[...] = y[:, 2 * D_MODEL:3 * D_MODEL]
    s_ref[...] = y[:, 3 * D_MODEL:4 * D_MODEL]


def _qkvs(node, Wcat, bcat):
    n = node.shape[0]
    grid = n // ROW_BLK
    row = lambda i: (i, 0)
    out = jax.ShapeDtypeStruct((n, D_MODEL), jnp.float32)
    return pl.pallas_call(
        _qkvs_body,
        grid=(grid,),
        in_specs=[
            pl.BlockSpec((ROW_BLK, D_MODEL), row),
            pl.BlockSpec((D_MODEL, 4 * D_MODEL), lambda i: (0, 0)),
            pl.BlockSpec((1, 4 * D_MODEL), lambda i: (0, 0)),
        ],
        out_specs=[pl.BlockSpec((ROW_BLK, D_MODEL), row)] * 4,
        out_shape=[out, out, out, out],
    )(node, Wcat, bcat)


def _ln(x, g, b, eps=1e-5):
    mu = jnp.mean(x, axis=-1, keepdims=True)
    var = jnp.mean((x - mu) ** 2, axis=-1, keepdims=True)
    return (x - mu) * lax.rsqrt(var + eps) * g + b


def _post_body(node_ref, agg_ref, skip_ref, wout_ref, bout_ref, g1_ref, be1_ref,
               w1_ref, b1_ref, w2_ref, b2_ref, g2_ref, be2_ref, o_ref):
    attn = agg_ref[...] + skip_ref[...]
    y = jnp.dot(attn, wout_ref[...], preferred_element_type=jnp.float32) + bout_ref[...]
    x1 = _ln(node_ref[...] + y, g1_ref[...], be1_ref[...])
    h = jnp.maximum(
        jnp.dot(x1, w1_ref[...], preferred_element_type=jnp.float32) + b1_ref[...],
        0.0,
    )
    ff = jnp.dot(h, w2_ref[...], preferred_element_type=jnp.float32) + b2_ref[...]
    o_ref[...] = _ln(x1 + ff, g2_ref[...], be2_ref[...])


def _post(node, agg, skip, Wout, bout, g1, be1, W1, b1, W2, b2, g2, be2):
    n = node.shape[0]
    grid = n // ROW_BLK
    row = lambda i: (i, 0)
    full = lambda i: (0, 0)
    return pl.pallas_call(
        _post_body,
        grid=(grid,),
        in_specs=[
            pl.BlockSpec((ROW_BLK, D_MODEL), row),
            pl.BlockSpec((ROW_BLK, D_MODEL), row),
            pl.BlockSpec((ROW_BLK, D_MODEL), row),
            pl.BlockSpec((D_MODEL, D_MODEL), full),
            pl.BlockSpec((1, D_MODEL), full),
            pl.BlockSpec((1, D_MODEL), full),
            pl.BlockSpec((1, D_MODEL), full),
            pl.BlockSpec((D_MODEL, D_FF), full),
            pl.BlockSpec((1, D_FF), full),
            pl.BlockSpec((D_FF, D_MODEL), full),
            pl.BlockSpec((1, D_MODEL), full),
            pl.BlockSpec((1, D_MODEL), full),
            pl.BlockSpec((1, D_MODEL), full),
        ],
        out_specs=pl.BlockSpec((ROW_BLK, D_MODEL), row),
        out_shape=jax.ShapeDtypeStruct((n, D_MODEL), jnp.float32),
    )(node, agg, skip, Wout, bout, g1, be1, W1, b1, W2, b2, g2, be2)


def _edge_logits(q, k, dst_a, src_a):
    mesh = plsc.VectorSubcoreMesh(core_axis_name="c", subcore_axis_name="s")

    HW = D_MODEL // 2
    HCOL = HW // N_HEADS

    @functools.partial(
        pl.kernel,
        mesh=mesh,
        compiler_params=pltpu.CompilerParams(use_tc_tiling_on_sc=False,
                                             needs_layout_passes=False),
        out_type=jax.ShapeDtypeStruct((E_PAD, 16), jnp.float32),
        scratch_types=[
            pltpu.VMEM((2, CH_A), jnp.int32),
            pltpu.VMEM((2, CH_A), jnp.int32),
            pltpu.VMEM((2, CH_A, HW), jnp.int32),
            pltpu.VMEM((2, CH_A, HW), jnp.int32),
            pltpu.VMEM((2, CH_A, 16), jnp.float32),
            pltpu.SemaphoreType.DMA,
            pltpu.SemaphoreType.DMA,
            pltpu.SemaphoreType.DMA,
            pltpu.SemaphoreType.DMA,
            pltpu.SemaphoreType.DMA,
            pltpu.SemaphoreType.DMA,
        ],
    )
    def kern(q_hbm, k_hbm, dst_hbm, src_hbm, p_hbm,
             dstb, srcb, qbuf, kbuf, pbuf, sq0, sk0, sq1, sk1, sp0, sp1):
        wid = lax.axis_index("s") * NC + lax.axis_index("c")
        lanes = lax.iota(jnp.int32, L)
        zero16 = jnp.zeros((L,), jnp.float32)
        zero_bf = jnp.zeros((2 * L,), jnp.bfloat16)
        himask = jnp.int32(-65536)
        sems = ((sq0, sk0, sp0), (sq1, sk1, sp1))

        def fire(ci, b):
            base = (wid * CHUNKS_A + ci) * CH_A
            pltpu.sync_copy(dst_hbm.at[pl.ds(base, CH_A)], dstb.at[b])
            pltpu.sync_copy(src_hbm.at[pl.ds(base, CH_A)], srcb.at[b])
            pltpu.async_copy(q_hbm.at[dstb.at[b]], qbuf.at[b], sems[b][0])
            pltpu.async_copy(k_hbm.at[srcb.at[b]], kbuf.at[b], sems[b][1])

        def wait_gathers(b):
            pltpu.make_async_copy(q_hbm.at[dstb.at[b]], qbuf.at[b],
                                  sems[b][0]).wait()
            pltpu.make_async_copy(k_hbm.at[srcb.at[b]], kbuf.at[b],
                                  sems[b][1]).wait()

        def wait_pwrite(ci, b):
            base = (wid * CHUNKS_A + ci) * CH_A
            pltpu.make_async_copy(pbuf.at[b],
                                  p_hbm.at[pl.ds(base, CH_A)],
                                  sems[b][2]).wait()

        def compute(ci, b):
            qb = qbuf.at[b]
            kb = kbuf.at[b]
            pb = pbuf.at[b]
            for g in range(CH_A // L):
                rows = g * L + lanes
                for h in range(N_HEADS):
                    hbase = jnp.full((L,), h * HCOL, jnp.int32)

                    def col_body(cc, acc):
                        acc_bf = zero_bf
                        for u in range(8):
                            colv = hbase + (cc * 8 + u)
                            qw = plsc.load_gather(qb, [rows, colv])
                            kw = plsc.load_gather(kb, [rows, colv])
                            acc_bf = acc_bf + (plsc.bitcast(qw, jnp.bfloat16)
                                               * plsc.bitcast(kw, jnp.bfloat16))
                        w = plsc.bitcast(acc_bf, jnp.int32)
                        hi = plsc.bitcast(w & himask, jnp.float32)
                        lo = plsc.bitcast(lax.shift_left(w, 16), jnp.float32)
                        return acc + hi + lo

                    acc = lax.fori_loop(0, HCOL // 8, col_body,
                                        jnp.zeros((L,), jnp.float32))
                    pv = jnp.exp(acc * INV_SQRT_C)
                    plsc.store_scatter(
                        pb, [rows, jnp.full((L,), h, jnp.int32)], pv)
                for h in range(N_HEADS, 16):
                    plsc.store_scatter(
                        pb, [rows, jnp.full((L,), h, jnp.int32)], zero16)
            base = (wid * CHUNKS_A + ci) * CH_A
            pltpu.async_copy(pbuf.at[b], p_hbm.at[pl.ds(base, CH_A)],
                             sems[b][2])

        fire(0, 0)

        def pair_body(i, carry):
            ci0 = 2 * i
            fire(ci0 + 1, 1)
            wait_gathers(0)

            @pl.when(i > 0)
            def _():
                wait_pwrite(ci0 - 2, 0)
            compute(ci0, 0)

            @pl.when(i < CHUNKS_A // 2 - 1)
            def _():
                fire(ci0 + 2, 0)
            wait_gathers(1)

            @pl.when(i > 0)
            def _():
                wait_pwrite(ci0 - 1, 1)
            compute(ci0 + 1, 1)
            return carry

        lax.fori_loop(0, CHUNKS_A // 2, pair_body, 0)
        wait_pwrite(CHUNKS_A - 2, 0)
        wait_pwrite(CHUNKS_A - 1, 1)

    return kern(q, k, dst_a, src_a)


def _edge_aggregate(v, p, dst, src):
    mesh = plsc.VectorSubcoreMesh(core_axis_name="c", subcore_axis_name="s")

    @functools.partial(
        pl.kernel,
        mesh=mesh,
        compiler_params=pltpu.CompilerParams(use_tc_tiling_on_sc=False,
                                             needs_layout_passes=False),
        out_type=jax.ShapeDtypeStruct((N_NODES, D_MODEL), jnp.float32),
        scratch_types=[
            pltpu.VMEM((NODE_BLK, D_MODEL), jnp.float32),
            pltpu.VMEM((NODE_BLK, 16), jnp.float32),
            pltpu.VMEM((PENDCAP, D_MODEL), jnp.float32),
            pltpu.VMEM((PENDCAP, 16), jnp.float32),
            pltpu.VMEM((PENDCAP,), jnp.int32),
            pltpu.VMEM((PENDCAP,), jnp.int32),
            pltpu.VMEM((PENDCAP + L,), jnp.int32),
            pltpu.VMEM((SCH,), jnp.int32),
            pltpu.VMEM((SCH,), jnp.int32),
            pltpu.SemaphoreType.DMA,
            pltpu.SemaphoreType.DMA,
        ],
    )
    def kern(v_hbm, p_hbm, dst_hbm, src_hbm, agg_hbm,
             aggv, sv, vbuf, pbv, pend_src, pend_e, pend_dl,
             dstc, srcc, sem1, sem2):
        wid = lax.axis_index("s") * NC + lax.axis_index("c")
        lanes = lax.iota(jnp.int32, L)
        lo = wid * NODE_BLK
        hi = lo + NODE_BLK
        zero16f = jnp.zeros((L,), jnp.float32)
        zero16i = jnp.zeros((L,), jnp.int32)

        def z_body(r, carry):
            for half in range(D_MODEL // L):
                aggv[r, pl.ds(half * L, L)] = zero16f
            sv[r, pl.ds(0, L)] = zero16f
            return carry
        lax.fori_loop(0, NODE_BLK, z_body, 0)
        for i in range(PENDCAP // L):
            pend_src[pl.ds(i * L, L)] = zero16i
            pend_e[pl.ds(i * L, L)] = zero16i

        def flush(cnt):
            cv = pltpu.async_copy(v_hbm.at[pend_src], vbuf, sem1)
            cp = pltpu.async_copy(p_hbm.at[pend_e], pbv, sem2)
            cv.wait()
            cp.wait()

            def acc_body(i, carry):
                dl = pend_dl[pl.ds(i, L)][0]
                prow = pbv[i]
                plsc.addupdate(sv.at[dl], prow)
                for h in range(N_HEADS):
                    ps = prow[h]
                    for half in range(2):
                        c0 = h * HEAD_DIM + half * L
                        vv = vbuf[i, pl.ds(c0, L)]
                        plsc.addupdate(aggv.at[dl, pl.ds(c0, L)], vv * ps)
                return carry

            lax.fori_loop(0, cnt, acc_body, 0)

        def chunk_body(ci, off):
            cb = ci * SCH
            pltpu.sync_copy(dst_hbm.at[pl.ds(cb, SCH)], dstc)
            pltpu.sync_copy(src_hbm.at[pl.ds(cb, SCH)], srcc)

            def group_body(gi, off):
                gb = gi * L
                dstv = dstc[pl.ds(gb, L)]
                srcv = srcc[pl.ds(gb, L)]
                m = (dstv >= lo) & (dstv < hi)
                cnt = plsc.all_reduce_population_count(m)[0]

                def do_flush(o):
                    flush(o)
                    return 0
                off = lax.cond(off + L > PENDCAP, do_flush, lambda o: o, off)
                plsc.store_compressed(pend_src.at[pl.ds(off, L)], srcv, mask=m)
                plsc.store_compressed(pend_e.at[pl.ds(off, L)],
                                      cb + gb + lanes, mask=m)
                plsc.store_compressed(pend_dl.at[pl.ds(off, L)],
                                      dstv - lo, mask=m)
                return off + cnt

            return lax.fori_loop(0, SCH // L, group_body, off)

        off = lax.fori_loop(0, NCHUNK_B, chunk_body, 0)
        lax.cond(off > 0, lambda o: (flush(o), 0)[1], lambda o: 0, off)

        def norm_body(r, carry):
            invrow = 1.0 / (sv[r] + 1e-16)
            for h in range(N_HEADS):
                inv = invrow[h]
                for half in range(2):
                    c0 = h * HEAD_DIM + half * L
                    aggv[r, pl.ds(c0, L)] = aggv[r, pl.ds(c0, L)] * inv
            return carry
        lax.fori_loop(0, NODE_BLK, norm_body, 0)

        @pl.when(wid < NW - 1)
        def _():
            pltpu.sync_copy(aggv, agg_hbm.at[pl.ds(lo, NODE_BLK)])

        @pl.when(wid == NW - 1)
        def _():
            tail = N_NODES - (NW - 1) * NODE_BLK
            pltpu.sync_copy(aggv.at[pl.ds(0, tail)],
                            agg_hbm.at[pl.ds(lo, tail)])

    return kern(v, p, dst, src)


def kernel(node, edge_index, Wq, bq, Wk, bk, Wv, bv, Wskip, bskip, Wout, bout,
           g1, be1, W1, b1, W2, b2, g2, be2):
    n = node.shape[0]
    src = edge_index[0].astype(jnp.int32)
    dst = edge_index[1].astype(jnp.int32)
    Wcat = jnp.concatenate([Wq, Wk, Wv, Wskip], axis=1)
    bcat = jnp.concatenate([bq, bk, bv, bskip]).reshape(1, 4 * D_MODEL)
    q, k, v, skip = _qkvs(node, Wcat, bcat)

    pad_e = E_PAD - N_EDGES
    src_a = jnp.pad(src, (0, pad_e))
    dst_a = jnp.pad(dst, (0, pad_e))
    q_i32 = lax.bitcast_convert_type(
        q.astype(jnp.bfloat16).reshape(n, D_MODEL // 2, 2), jnp.int32)
    k_i32 = lax.bitcast_convert_type(
        k.astype(jnp.bfloat16).reshape(n, D_MODEL // 2, 2), jnp.int32)
    p = _edge_logits(q_i32, k_i32, dst_a, src_a)
    agg = _edge_aggregate(v, p, dst, src)

    return _post(node, agg, skip, Wout, bout.reshape(1, -1), g1.reshape(1, -1),
                 be1.reshape(1, -1), W1, b1.reshape(1, -1), W2, b2.reshape(1, -1),
                 g2.reshape(1, -1), be2.reshape(1, -1))


# speedup vs baseline: 1.4395x; 1.1099x over previous
import functools

import jax
import jax.numpy as jnp
from jax import lax
from jax.experimental import pallas as pl
from jax.experimental.pallas import tpu as pltpu
from jax.experimental.pallas import tpu_sc as plsc

D_MODEL = 256
N_HEADS = 8
HEAD_DIM = D_MODEL // N_HEADS
D_FF = 1024
ROW_BLK = 400

NC = 2
NS = 16
L = 16
NW = NC * NS

N_NODES = 10000
N_EDGES = 160000
CH_A = 128
CHUNKS_A = 40
E_PAD = NW * CHUNKS_A * CH_A
NODE_BLK = 313
SCH = 1600
NCHUNK_B = N_EDGES // SCH
PENDCAP = 128
FLUSH_AT = 112
INV_SQRT_C = 0.17677669529663687


def _qkvs_body(x_ref, w_ref, b_ref, q_ref, k_ref, v_ref, s_ref):
    y = (
        jnp.dot(x_ref[...], w_ref[...], preferred_element_type=jnp.float32)
        + b_ref[...]
    )
    q_ref[...] = y[:, 0:D_MODEL]
    k_ref[...] = y[:, D_MODEL:2 * D_MODEL]
    v_ref[...] = y[:, 2 * D_MODEL:3 * D_MODEL]
    s_ref[...] = y[:, 3 * D_MODEL:4 * D_MODEL]


def _qkvs(node, Wcat, bcat):
    n = node.shape[0]
    grid = n // ROW_BLK
    row = lambda i: (i, 0)
    out = jax.ShapeDtypeStruct((n, D_MODEL), jnp.float32)
    return pl.pallas_call(
        _qkvs_body,
        grid=(grid,),
        in_specs=[
            pl.BlockSpec((ROW_BLK, D_MODEL), row),
            pl.BlockSpec((D_MODEL, 4 * D_MODEL), lambda i: (0, 0)),
            pl.BlockSpec((1, 4 * D_MODEL), lambda i: (0, 0)),
        ],
        out_specs=[pl.BlockSpec((ROW_BLK, D_MODEL), row)] * 4,
        out_shape=[out, out, out, out],
    )(node, Wcat, bcat)


def _ln(x, g, b, eps=1e-5):
    mu = jnp.mean(x, axis=-1, keepdims=True)
    var = jnp.mean((x - mu) ** 2, axis=-1, keepdims=True)
    return (x - mu) * lax.rsqrt(var + eps) * g + b


def _post_body(node_ref, agg_ref, skip_ref, wout_ref, bout_ref, g1_ref, be1_ref,
               w1_ref, b1_ref, w2_ref, b2_ref, g2_ref, be2_ref, o_ref):
    attn = agg_ref[...] + skip_ref[...]
    y = jnp.dot(attn, wout_ref[...], preferred_element_type=jnp.float32) + bout_ref[...]
    x1 = _ln(node_ref[...] + y, g1_ref[...], be1_ref[...])
    h = jnp.maximum(
        jnp.dot(x1, w1_ref[...], preferred_element_type=jnp.float32) + b1_ref[...],
        0.0,
    )
    ff = jnp.dot(h, w2_ref[...], preferred_element_type=jnp.float32) + b2_ref[...]
    o_ref[...] = _ln(x1 + ff, g2_ref[...], be2_ref[...])


def _post(node, agg, skip, Wout, bout, g1, be1, W1, b1, W2, b2, g2, be2):
    n = node.shape[0]
    grid = n // ROW_BLK
    row = lambda i: (i, 0)
    full = lambda i: (0, 0)
    return pl.pallas_call(
        _post_body,
        grid=(grid,),
        in_specs=[
            pl.BlockSpec((ROW_BLK, D_MODEL), row),
            pl.BlockSpec((ROW_BLK, D_MODEL), row),
            pl.BlockSpec((ROW_BLK, D_MODEL), row),
            pl.BlockSpec((D_MODEL, D_MODEL), full),
            pl.BlockSpec((1, D_MODEL), full),
            pl.BlockSpec((1, D_MODEL), full),
            pl.BlockSpec((1, D_MODEL), full),
            pl.BlockSpec((D_MODEL, D_FF), full),
            pl.BlockSpec((1, D_FF), full),
            pl.BlockSpec((D_FF, D_MODEL), full),
            pl.BlockSpec((1, D_MODEL), full),
            pl.BlockSpec((1, D_MODEL), full),
            pl.BlockSpec((1, D_MODEL), full),
        ],
        out_specs=pl.BlockSpec((ROW_BLK, D_MODEL), row),
        out_shape=jax.ShapeDtypeStruct((n, D_MODEL), jnp.float32),
    )(node, agg, skip, Wout, bout, g1, be1, W1, b1, W2, b2, g2, be2)


def _edge_logits(q, k, dst_a, src_a):
    mesh = plsc.VectorSubcoreMesh(core_axis_name="c", subcore_axis_name="s")

    HW = D_MODEL // 2
    HCOL = HW // N_HEADS

    @functools.partial(
        pl.kernel,
        mesh=mesh,
        compiler_params=pltpu.CompilerParams(use_tc_tiling_on_sc=False,
                                             needs_layout_passes=False),
        out_type=jax.ShapeDtypeStruct((E_PAD, 16), jnp.float32),
        scratch_types=[
            pltpu.VMEM((2, CH_A), jnp.int32),
            pltpu.VMEM((2, CH_A), jnp.int32),
            pltpu.VMEM((2, CH_A, HW), jnp.int32),
            pltpu.VMEM((2, CH_A, HW), jnp.int32),
            pltpu.VMEM((2, CH_A, 16), jnp.float32),
            pltpu.SemaphoreType.DMA,
            pltpu.SemaphoreType.DMA,
            pltpu.SemaphoreType.DMA,
            pltpu.SemaphoreType.DMA,
            pltpu.SemaphoreType.DMA,
            pltpu.SemaphoreType.DMA,
        ],
    )
    def kern(q_hbm, k_hbm, dst_hbm, src_hbm, p_hbm,
             dstb, srcb, qbuf, kbuf, pbuf, sq0, sk0, sq1, sk1, sp0, sp1):
        wid = lax.axis_index("s") * NC + lax.axis_index("c")
        lanes = lax.iota(jnp.int32, L)
        zero16 = jnp.zeros((L,), jnp.float32)
        zero_bf = jnp.zeros((2 * L,), jnp.bfloat16)
        himask = jnp.int32(-65536)
        sems = ((sq0, sk0, sp0), (sq1, sk1, sp1))

        def fire(ci, b):
            base = (wid * CHUNKS_A + ci) * CH_A
            pltpu.sync_copy(dst_hbm.at[pl.ds(base, CH_A)], dstb.at[b])
            pltpu.sync_copy(src_hbm.at[pl.ds(base, CH_A)], srcb.at[b])
            pltpu.async_copy(q_hbm.at[dstb.at[b]], qbuf.at[b], sems[b][0])
            pltpu.async_copy(k_hbm.at[srcb.at[b]], kbuf.at[b], sems[b][1])

        def wait_gathers(b):
            pltpu.make_async_copy(q_hbm.at[dstb.at[b]], qbuf.at[b],
                                  sems[b][0]).wait()
            pltpu.make_async_copy(k_hbm.at[srcb.at[b]], kbuf.at[b],
                                  sems[b][1]).wait()

        def wait_pwrite(ci, b):
            base = (wid * CHUNKS_A + ci) * CH_A
            pltpu.make_async_copy(pbuf.at[b],
                                  p_hbm.at[pl.ds(base, CH_A)],
                                  sems[b][2]).wait()

        def compute(ci, b):
            qb = qbuf.at[b]
            kb = kbuf.at[b]
            pb = pbuf.at[b]
            for g in range(CH_A // L):
                rows = g * L + lanes
                for h in range(N_HEADS):
                    hbase = jnp.full((L,), h * HCOL, jnp.int32)

                    def col_body(cc, acc):
                        acc_bf = zero_bf
                        for u in range(8):
                            colv = hbase + (cc * 8 + u)
                            qw = plsc.load_gather(qb, [rows, colv])
                            kw = plsc.load_gather(kb, [rows, colv])
                            acc_bf = acc_bf + (plsc.bitcast(qw, jnp.bfloat16)
                                               * plsc.bitcast(kw, jnp.bfloat16))
                        w = plsc.bitcast(acc_bf, jnp.int32)
                        hi = plsc.bitcast(w & himask, jnp.float32)
                        lo = plsc.bitcast(lax.shift_left(w, 16), jnp.float32)
                        return acc + hi + lo

                    acc = lax.fori_loop(0, HCOL // 8, col_body,
                                        jnp.zeros((L,), jnp.float32))
                    pv = jnp.exp(acc * INV_SQRT_C)
                    plsc.store_scatter(
                        pb, [rows, jnp.full((L,), h, jnp.int32)], pv)
                for h in range(N_HEADS, 16):
                    plsc.store_scatter(
                        pb, [rows, jnp.full((L,), h, jnp.int32)], zero16)
            base = (wid * CHUNKS_A + ci) * CH_A
            pltpu.async_copy(pbuf.at[b], p_hbm.at[pl.ds(base, CH_A)],
                             sems[b][2])

        fire(0, 0)

        def pair_body(i, carry):
            ci0 = 2 * i
            fire(ci0 + 1, 1)
            wait_gathers(0)

            @pl.when(i > 0)
            def _():
                wait_pwrite(ci0 - 2, 0)
            compute(ci0, 0)

            @pl.when(i < CHUNKS_A // 2 - 1)
            def _():
                fire(ci0 + 2, 0)
            wait_gathers(1)

            @pl.when(i > 0)
            def _():
                wait_pwrite(ci0 - 1, 1)
            compute(ci0 + 1, 1)
            return carry

        lax.fori_loop(0, CHUNKS_A // 2, pair_body, 0)
        wait_pwrite(CHUNKS_A - 2, 0)
        wait_pwrite(CHUNKS_A - 1, 1)

    return kern(q, k, dst_a, src_a)


def _edge_aggregate(v, p, dst, src):
    mesh = plsc.VectorSubcoreMesh(core_axis_name="c", subcore_axis_name="s")

    @functools.partial(
        pl.kernel,
        mesh=mesh,
        compiler_params=pltpu.CompilerParams(use_tc_tiling_on_sc=False,
                                             needs_layout_passes=False),
        out_type=jax.ShapeDtypeStruct((N_NODES, D_MODEL), jnp.float32),
        scratch_types=[
            pltpu.VMEM((NODE_BLK, D_MODEL), jnp.float32),
            pltpu.VMEM((NODE_BLK, 16), jnp.float32),
            pltpu.VMEM((PENDCAP, D_MODEL), jnp.float32),
            pltpu.VMEM((PENDCAP, 16), jnp.float32),
            pltpu.VMEM((PENDCAP,), jnp.int32),
            pltpu.VMEM((PENDCAP,), jnp.int32),
            pltpu.VMEM((PENDCAP + L,), jnp.int32),
            pltpu.VMEM((2, SCH), jnp.int32),
            pltpu.VMEM((2, SCH), jnp.int32),
            pltpu.SemaphoreType.DMA,
            pltpu.SemaphoreType.DMA,
            pltpu.SemaphoreType.DMA,
            pltpu.SemaphoreType.DMA,
        ],
    )
    def kern(v_hbm, p_hbm, dst_hbm, src_hbm, agg_hbm,
             aggv, sv, vbuf, pbv, pend_src, pend_e, pend_dl,
             dstc, srcc, sem1, sem2, sd0, sd1):
        wid = lax.axis_index("s") * NC + lax.axis_index("c")
        lanes = lax.iota(jnp.int32, L)
        lo = wid * NODE_BLK
        hi = lo + NODE_BLK
        zero16f = jnp.zeros((L,), jnp.float32)
        zero16i = jnp.zeros((L,), jnp.int32)

        def z_body(r, carry):
            for half in range(D_MODEL // L):
                aggv[r, pl.ds(half * L, L)] = zero16f
            sv[r, pl.ds(0, L)] = zero16f
            return carry
        lax.fori_loop(0, NODE_BLK, z_body, 0)
        for i in range(PENDCAP // L):
            pend_src[pl.ds(i * L, L)] = zero16i
            pend_e[pl.ds(i * L, L)] = zero16i

        def flush(cnt):
            cv = pltpu.async_copy(v_hbm.at[pend_src], vbuf, sem1)
            cp = pltpu.async_copy(p_hbm.at[pend_e], pbv, sem2)
            cv.wait()
            cp.wait()

            def acc_body(i, carry):
                dl = pend_dl[pl.ds(i, L)][0]
                prow = pbv[i]
                plsc.addupdate(sv.at[dl], prow)
                for h in range(N_HEADS):
                    ps = prow[h]
                    for half in range(2):
                        c0 = h * HEAD_DIM + half * L
                        vv = vbuf[i, pl.ds(c0, L)]
                        plsc.addupdate(aggv.at[dl, pl.ds(c0, L)], vv * ps)
                return carry

            lax.fori_loop(0, cnt, acc_body, 0)

        sdc = (sd0, sd1)

        def fire_scan(ci, b):
            cb = ci * SCH
            pltpu.async_copy(dst_hbm.at[pl.ds(cb, SCH)], dstc.at[b], sdc[b])
            pltpu.async_copy(src_hbm.at[pl.ds(cb, SCH)], srcc.at[b], sdc[b])

        def wait_scan(ci, b):
            cb = ci * SCH
            pltpu.make_async_copy(dst_hbm.at[pl.ds(cb, SCH)], dstc.at[b],
                                  sdc[b]).wait()
            pltpu.make_async_copy(src_hbm.at[pl.ds(cb, SCH)], srcc.at[b],
                                  sdc[b]).wait()

        def scan_chunk(ci, b, off):
            cb = ci * SCH

            def group_body(gi, off):
                gb = gi * L
                dstv = dstc[b, pl.ds(gb, L)]
                srcv = srcc[b, pl.ds(gb, L)]
                m = (dstv >= lo) & (dstv < hi)
                cnt = plsc.all_reduce_population_count(m)[0]

                def do_flush(o):
                    flush(o)
                    return 0
                off = lax.cond(off + L > PENDCAP, do_flush, lambda o: o, off)
                plsc.store_compressed(pend_src.at[pl.ds(off, L)], srcv, mask=m)
                plsc.store_compressed(pend_e.at[pl.ds(off, L)],
                                      cb + gb + lanes, mask=m)
                plsc.store_compressed(pend_dl.at[pl.ds(off, L)],
                                      dstv - lo, mask=m)
                return off + cnt

            return lax.fori_loop(0, SCH // L, group_body, off)

        fire_scan(0, 0)

        def pair_body(i, off):
            ci0 = 2 * i
            fire_scan(ci0 + 1, 1)
            wait_scan(ci0, 0)
            off = scan_chunk(ci0, 0, off)

            @pl.when(i < NCHUNK_B // 2 - 1)
            def _():
                fire_scan(ci0 + 2, 0)
            wait_scan(ci0 + 1, 1)
            return scan_chunk(ci0 + 1, 1, off)

        off = lax.fori_loop(0, NCHUNK_B // 2, pair_body, 0)
        lax.cond(off > 0, lambda o: (flush(o), 0)[1], lambda o: 0, off)

        def norm_body(r, carry):
            invrow = 1.0 / (sv[r] + 1e-16)
            for h in range(N_HEADS):
                inv = invrow[h]
                for half in range(2):
                    c0 = h * HEAD_DIM + half * L
                    aggv[r, pl.ds(c0, L)] = aggv[r, pl.ds(c0, L)] * inv
            return carry
        lax.fori_loop(0, NODE_BLK, norm_body, 0)

        @pl.when(wid < NW - 1)
        def _():
            pltpu.sync_copy(aggv, agg_hbm.at[pl.ds(lo, NODE_BLK)])

        @pl.when(wid == NW - 1)
        def _():
            tail = N_NODES - (NW - 1) * NODE_BLK
            pltpu.sync_copy(aggv.at[pl.ds(0, tail)],
                            agg_hbm.at[pl.ds(lo, tail)])

    return kern(v, p, dst, src)


def kernel(node, edge_index, Wq, bq, Wk, bk, Wv, bv, Wskip, bskip, Wout, bout,
           g1, be1, W1, b1, W2, b2, g2, be2):
    n = node.shape[0]
    src = edge_index[0].astype(jnp.int32)
    dst = edge_index[1].astype(jnp.int32)
    Wcat = jnp.concatenate([Wq, Wk, Wv, Wskip], axis=1)
    bcat = jnp.concatenate([bq, bk, bv, bskip]).reshape(1, 4 * D_MODEL)
    q, k, v, skip = _qkvs(node, Wcat, bcat)

    pad_e = E_PAD - N_EDGES
    src_a = jnp.pad(src, (0, pad_e))
    dst_a = jnp.pad(dst, (0, pad_e))
    q_i32 = lax.bitcast_convert_type(
        q.astype(jnp.bfloat16).reshape(n, D_MODEL // 2, 2), jnp.int32)
    k_i32 = lax.bitcast_convert_type(
        k.astype(jnp.bfloat16).reshape(n, D_MODEL // 2, 2), jnp.int32)
    p = _edge_logits(q_i32, k_i32, dst_a, src_a)
    agg = _edge_aggregate(v, p, dst, src)

    return _post(node, agg, skip, Wout, bout.reshape(1, -1), g1.reshape(1, -1),
                 be1.reshape(1, -1), W1, b1.reshape(1, -1), W2, b2.reshape(1, -1),
                 g2.reshape(1, -1), be2.reshape(1, -1))


# speedup vs baseline: 1.4654x; 1.0180x over previous
import functools

import jax
import jax.numpy as jnp
from jax import lax
from jax.experimental import pallas as pl
from jax.experimental.pallas import tpu as pltpu
from jax.experimental.pallas import tpu_sc as plsc

D_MODEL = 256
N_HEADS = 8
HEAD_DIM = D_MODEL // N_HEADS
D_FF = 1024
ROW_BLK = 400

NC = 2
NS = 16
L = 16
NW = NC * NS

N_NODES = 10000
N_EDGES = 160000
CH_A = 128
CHUNKS_A = 40
E_PAD = NW * CHUNKS_A * CH_A
NODE_BLK = 313
SCH = 1600
NCHUNK_B = N_EDGES // SCH
PENDCAP = 128
FLUSH_AT = 112
INV_SQRT_C = 0.17677669529663687


def _qkvs_body(x_ref, w_ref, b_ref, q_ref, k_ref, v_ref, s_ref):
    y = (
        jnp.dot(x_ref[...], w_ref[...], preferred_element_type=jnp.float32)
        + b_ref[...]
    )
    q_ref[...] = y[:, 0:D_MODEL]
    k_ref[...] = y[:, D_MODEL:2 * D_MODEL]
    v_ref[...] = y[:, 2 * D_MODEL:3 * D_MODEL]
    s_ref[...] = y[:, 3 * D_MODEL:4 * D_MODEL]


def _qkvs(node, Wcat, bcat):
    n = node.shape[0]
    grid = n // ROW_BLK
    row = lambda i: (i, 0)
    out = jax.ShapeDtypeStruct((n, D_MODEL), jnp.float32)
    return pl.pallas_call(
        _qkvs_body,
        grid=(grid,),
        in_specs=[
            pl.BlockSpec((ROW_BLK, D_MODEL), row),
            pl.BlockSpec((D_MODEL, 4 * D_MODEL), lambda i: (0, 0)),
            pl.BlockSpec((1, 4 * D_MODEL), lambda i: (0, 0)),
        ],
        out_specs=[pl.BlockSpec((ROW_BLK, D_MODEL), row)] * 4,
        out_shape=[out, out, out, out],
    )(node, Wcat, bcat)


def _ln(x, g, b, eps=1e-5):
    mu = jnp.mean(x, axis=-1, keepdims=True)
    var = jnp.mean((x - mu) ** 2, axis=-1, keepdims=True)
    return (x - mu) * lax.rsqrt(var + eps) * g + b


def _post_body(node_ref, agg_ref, skip_ref, wout_ref, bout_ref, g1_ref, be1_ref,
               w1_ref, b1_ref, w2_ref, b2_ref, g2_ref, be2_ref, o_ref):
    attn = agg_ref[...] + skip_ref[...]
    y = jnp.dot(attn, wout_ref[...], preferred_element_type=jnp.float32) + bout_ref[...]
    x1 = _ln(node_ref[...] + y, g1_ref[...], be1_ref[...])
    h = jnp.maximum(
        jnp.dot(x1, w1_ref[...], preferred_element_type=jnp.float32) + b1_ref[...],
        0.0,
    )
    ff = jnp.dot(h, w2_ref[...], preferred_element_type=jnp.float32) + b2_ref[...]
    o_ref[...] = _ln(x1 + ff, g2_ref[...], be2_ref[...])


def _post(node, agg, skip, Wout, bout, g1, be1, W1, b1, W2, b2, g2, be2):
    n = node.shape[0]
    grid = n // ROW_BLK
    row = lambda i: (i, 0)
    full = lambda i: (0, 0)
    return pl.pallas_call(
        _post_body,
        grid=(grid,),
        in_specs=[
            pl.BlockSpec((ROW_BLK, D_MODEL), row),
            pl.BlockSpec((ROW_BLK, D_MODEL), row),
            pl.BlockSpec((ROW_BLK, D_MODEL), row),
            pl.BlockSpec((D_MODEL, D_MODEL), full),
            pl.BlockSpec((1, D_MODEL), full),
            pl.BlockSpec((1, D_MODEL), full),
            pl.BlockSpec((1, D_MODEL), full),
            pl.BlockSpec((D_MODEL, D_FF), full),
            pl.BlockSpec((1, D_FF), full),
            pl.BlockSpec((D_FF, D_MODEL), full),
            pl.BlockSpec((1, D_MODEL), full),
            pl.BlockSpec((1, D_MODEL), full),
            pl.BlockSpec((1, D_MODEL), full),
        ],
        out_specs=pl.BlockSpec((ROW_BLK, D_MODEL), row),
        out_shape=jax.ShapeDtypeStruct((n, D_MODEL), jnp.float32),
    )(node, agg, skip, Wout, bout, g1, be1, W1, b1, W2, b2, g2, be2)


def _edge_logits(q, k, dst_a, src_a):
    mesh = plsc.VectorSubcoreMesh(core_axis_name="c", subcore_axis_name="s")

    HW = D_MODEL // 2
    HCOL = HW // N_HEADS

    @functools.partial(
        pl.kernel,
        mesh=mesh,
        compiler_params=pltpu.CompilerParams(use_tc_tiling_on_sc=False,
                                             needs_layout_passes=False),
        out_type=jax.ShapeDtypeStruct((E_PAD, 16), jnp.float32),
        scratch_types=[
            pltpu.VMEM((CHUNKS_A * CH_A,), jnp.int32),
            pltpu.VMEM((CHUNKS_A * CH_A,), jnp.int32),
            pltpu.VMEM((2, CH_A, HW), jnp.int32),
            pltpu.VMEM((2, CH_A, HW), jnp.int32),
            pltpu.VMEM((2, CH_A, 16), jnp.float32),
            pltpu.SemaphoreType.DMA,
            pltpu.SemaphoreType.DMA,
            pltpu.SemaphoreType.DMA,
            pltpu.SemaphoreType.DMA,
            pltpu.SemaphoreType.DMA,
            pltpu.SemaphoreType.DMA,
        ],
    )
    def kern(q_hbm, k_hbm, dst_hbm, src_hbm, p_hbm,
             dstb, srcb, qbuf, kbuf, pbuf, sq0, sk0, sq1, sk1, sp0, sp1):
        wid = lax.axis_index("s") * NC + lax.axis_index("c")
        lanes = lax.iota(jnp.int32, L)
        zero16 = jnp.zeros((L,), jnp.float32)
        zero_bf = jnp.zeros((2 * L,), jnp.bfloat16)
        himask = jnp.int32(-65536)
        sems = ((sq0, sk0, sp0), (sq1, sk1, sp1))

        wbase = wid * CHUNKS_A * CH_A
        pltpu.sync_copy(dst_hbm.at[pl.ds(wbase, CHUNKS_A * CH_A)], dstb)
        pltpu.sync_copy(src_hbm.at[pl.ds(wbase, CHUNKS_A * CH_A)], srcb)

        def fire(ci, b):
            cb = ci * CH_A
            pltpu.async_copy(q_hbm.at[dstb.at[pl.ds(cb, CH_A)]],
                             qbuf.at[b], sems[b][0])
            pltpu.async_copy(k_hbm.at[srcb.at[pl.ds(cb, CH_A)]],
                             kbuf.at[b], sems[b][1])

        def wait_gathers(ci, b):
            cb = ci * CH_A
            pltpu.make_async_copy(q_hbm.at[dstb.at[pl.ds(cb, CH_A)]],
                                  qbuf.at[b], sems[b][0]).wait()
            pltpu.make_async_copy(k_hbm.at[srcb.at[pl.ds(cb, CH_A)]],
                                  kbuf.at[b], sems[b][1]).wait()

        def wait_pwrite(ci, b):
            base = (wid * CHUNKS_A + ci) * CH_A
            pltpu.make_async_copy(pbuf.at[b],
                                  p_hbm.at[pl.ds(base, CH_A)],
                                  sems[b][2]).wait()

        def compute(ci, b):
            qb = qbuf.at[b]
            kb = kbuf.at[b]
            pb = pbuf.at[b]
            for g in range(CH_A // L):
                rows = g * L + lanes
                for h in range(N_HEADS):
                    hbase = jnp.full((L,), h * HCOL, jnp.int32)

                    def col_body(cc, acc):
                        acc_bf = zero_bf
                        for u in range(8):
                            colv = hbase + (cc * 8 + u)
                            qw = plsc.load_gather(qb, [rows, colv])
                            kw = plsc.load_gather(kb, [rows, colv])
                            acc_bf = acc_bf + (plsc.bitcast(qw, jnp.bfloat16)
                                               * plsc.bitcast(kw, jnp.bfloat16))
                        w = plsc.bitcast(acc_bf, jnp.int32)
                        hi = plsc.bitcast(w & himask, jnp.float32)
                        lo = plsc.bitcast(lax.shift_left(w, 16), jnp.float32)
                        return acc + hi + lo

                    acc = lax.fori_loop(0, HCOL // 8, col_body,
                                        jnp.zeros((L,), jnp.float32))
                    pv = jnp.exp(acc * INV_SQRT_C)
                    plsc.store_scatter(
                        pb, [rows, jnp.full((L,), h, jnp.int32)], pv)
                for h in range(N_HEADS, 16):
                    plsc.store_scatter(
                        pb, [rows, jnp.full((L,), h, jnp.int32)], zero16)
            base = (wid * CHUNKS_A + ci) * CH_A
            pltpu.async_copy(pbuf.at[b], p_hbm.at[pl.ds(base, CH_A)],
                             sems[b][2])

        fire(0, 0)

        def pair_body(i, carry):
            ci0 = 2 * i
            fire(ci0 + 1, 1)
            wait_gathers(ci0, 0)

            @pl.when(i > 0)
            def _():
                wait_pwrite(ci0 - 2, 0)
            compute(ci0, 0)

            @pl.when(i < CHUNKS_A // 2 - 1)
            def _():
                fire(ci0 + 2, 0)
            wait_gathers(ci0 + 1, 1)

            @pl.when(i > 0)
            def _():
                wait_pwrite(ci0 - 1, 1)
            compute(ci0 + 1, 1)
            return carry

        lax.fori_loop(0, CHUNKS_A // 2, pair_body, 0)
        wait_pwrite(CHUNKS_A - 2, 0)
        wait_pwrite(CHUNKS_A - 1, 1)

    return kern(q, k, dst_a, src_a)


def _edge_aggregate(v, p, dst, src):
    mesh = plsc.VectorSubcoreMesh(core_axis_name="c", subcore_axis_name="s")

    @functools.partial(
        pl.kernel,
        mesh=mesh,
        compiler_params=pltpu.CompilerParams(use_tc_tiling_on_sc=False,
                                             needs_layout_passes=False),
        out_type=jax.ShapeDtypeStruct((N_NODES, D_MODEL), jnp.float32),
        scratch_types=[
            pltpu.VMEM((NODE_BLK, D_MODEL), jnp.float32),
            pltpu.VMEM((NODE_BLK, 16), jnp.float32),
            pltpu.VMEM((PENDCAP, D_MODEL), jnp.float32),
            pltpu.VMEM((PENDCAP, 16), jnp.float32),
            pltpu.VMEM((PENDCAP,), jnp.int32),
            pltpu.VMEM((PENDCAP,), jnp.int32),
            pltpu.VMEM((PENDCAP + L,), jnp.int32),
            pltpu.VMEM((2, SCH), jnp.int32),
            pltpu.VMEM((2, SCH), jnp.int32),
            pltpu.SemaphoreType.DMA,
            pltpu.SemaphoreType.DMA,
            pltpu.SemaphoreType.DMA,
            pltpu.SemaphoreType.DMA,
        ],
    )
    def kern(v_hbm, p_hbm, dst_hbm, src_hbm, agg_hbm,
             aggv, sv, vbuf, pbv, pend_src, pend_e, pend_dl,
             dstc, srcc, sem1, sem2, sd0, sd1):
        wid = lax.axis_index("s") * NC + lax.axis_index("c")
        lanes = lax.iota(jnp.int32, L)
        lo = wid * NODE_BLK
        hi = lo + NODE_BLK
        zero16f = jnp.zeros((L,), jnp.float32)
        zero16i = jnp.zeros((L,), jnp.int32)

        def z_body(r, carry):
            for half in range(D_MODEL // L):
                aggv[r, pl.ds(half * L, L)] = zero16f
            sv[r, pl.ds(0, L)] = zero16f
            return carry
        lax.fori_loop(0, NODE_BLK, z_body, 0)
        for i in range(PENDCAP // L):
            pend_src[pl.ds(i * L, L)] = zero16i
            pend_e[pl.ds(i * L, L)] = zero16i

        def flush(cnt):
            cv = pltpu.async_copy(v_hbm.at[pend_src], vbuf, sem1)
            cp = pltpu.async_copy(p_hbm.at[pend_e], pbv, sem2)
            cv.wait()
            cp.wait()

            def acc_body(i, carry):
                dl = pend_dl[pl.ds(i, L)][0]
                prow = pbv[i]
                plsc.addupdate(sv.at[dl], prow)
                for h in range(N_HEADS):
                    ps = prow[h]
                    for half in range(2):
                        c0 = h * HEAD_DIM + half * L
                        vv = vbuf[i, pl.ds(c0, L)]
                        plsc.addupdate(aggv.at[dl, pl.ds(c0, L)], vv * ps)
                return carry

            lax.fori_loop(0, cnt, acc_body, 0)

        sdc = (sd0, sd1)

        def fire_scan(ci, b):
            cb = ci * SCH
            pltpu.async_copy(dst_hbm.at[pl.ds(cb, SCH)], dstc.at[b], sdc[b])
            pltpu.async_copy(src_hbm.at[pl.ds(cb, SCH)], srcc.at[b], sdc[b])

        def wait_scan(ci, b):
            cb = ci * SCH
            pltpu.make_async_copy(dst_hbm.at[pl.ds(cb, SCH)], dstc.at[b],
                                  sdc[b]).wait()
            pltpu.make_async_copy(src_hbm.at[pl.ds(cb, SCH)], srcc.at[b],
                                  sdc[b]).wait()

        def scan_chunk(ci, b, off):
            cb = ci * SCH

            def group_body(gi, off):
                gb = gi * L
                dstv = dstc[b, pl.ds(gb, L)]
                srcv = srcc[b, pl.ds(gb, L)]
                m = (dstv >= lo) & (dstv < hi)
                cnt = plsc.all_reduce_population_count(m)[0]

                def do_flush(o):
                    flush(o)
                    return 0
                off = lax.cond(off + L > PENDCAP, do_flush, lambda o: o, off)
                plsc.store_compressed(pend_src.at[pl.ds(off, L)], srcv, mask=m)
                plsc.store_compressed(pend_e.at[pl.ds(off, L)],
                                      cb + gb + lanes, mask=m)
                plsc.store_compressed(pend_dl.at[pl.ds(off, L)],
                                      dstv - lo, mask=m)
                return off + cnt

            return lax.fori_loop(0, SCH // L, group_body, off)

        fire_scan(0, 0)

        def pair_body(i, off):
            ci0 = 2 * i
            fire_scan(ci0 + 1, 1)
            wait_scan(ci0, 0)
            off = scan_chunk(ci0, 0, off)

            @pl.when(i < NCHUNK_B // 2 - 1)
            def _():
                fire_scan(ci0 + 2, 0)
            wait_scan(ci0 + 1, 1)
            return scan_chunk(ci0 + 1, 1, off)

        off = lax.fori_loop(0, NCHUNK_B // 2, pair_body, 0)
        lax.cond(off > 0, lambda o: (flush(o), 0)[1], lambda o: 0, off)

        def norm_body(r, carry):
            invrow = 1.0 / (sv[r] + 1e-16)
            for h in range(N_HEADS):
                inv = invrow[h]
                for half in range(2):
                    c0 = h * HEAD_DIM + half * L
                    aggv[r, pl.ds(c0, L)] = aggv[r, pl.ds(c0, L)] * inv
            return carry
        lax.fori_loop(0, NODE_BLK, norm_body, 0)

        @pl.when(wid < NW - 1)
        def _():
            pltpu.sync_copy(aggv, agg_hbm.at[pl.ds(lo, NODE_BLK)])

        @pl.when(wid == NW - 1)
        def _():
            tail = N_NODES - (NW - 1) * NODE_BLK
            pltpu.sync_copy(aggv.at[pl.ds(0, tail)],
                            agg_hbm.at[pl.ds(lo, tail)])

    return kern(v, p, dst, src)


def kernel(node, edge_index, Wq, bq, Wk, bk, Wv, bv, Wskip, bskip, Wout, bout,
           g1, be1, W1, b1, W2, b2, g2, be2):
    n = node.shape[0]
    src = edge_index[0].astype(jnp.int32)
    dst = edge_index[1].astype(jnp.int32)
    Wcat = jnp.concatenate([Wq, Wk, Wv, Wskip], axis=1)
    bcat = jnp.concatenate([bq, bk, bv, bskip]).reshape(1, 4 * D_MODEL)
    q, k, v, skip = _qkvs(node, Wcat, bcat)

    pad_e = E_PAD - N_EDGES
    src_a = jnp.pad(src, (0, pad_e))
    dst_a = jnp.pad(dst, (0, pad_e))
    q_i32 = lax.bitcast_convert_type(
        q.astype(jnp.bfloat16).reshape(n, D_MODEL // 2, 2), jnp.int32)
    k_i32 = lax.bitcast_convert_type(
        k.astype(jnp.bfloat16).reshape(n, D_MODEL // 2, 2), jnp.int32)
    p = _edge_logits(q_i32, k_i32, dst_a, src_a)
    agg = _edge_aggregate(v, p, dst, src)

    return _post(node, agg, skip, Wout, bout.reshape(1, -1), g1.reshape(1, -1),
                 be1.reshape(1, -1), W1, b1.reshape(1, -1), W2, b2.reshape(1, -1),
                 g2.reshape(1, -1), be2.reshape(1, -1))
